# baseline (speedup 1.0000x reference)
import jax
import jax.numpy as jnp
from jax.experimental import pallas as pl
from jax.experimental.pallas import tpu as pltpu

_B, _C, _H, _W = 64, 3, 256, 256
_CH = _C * _H
_RPC = 192
_NCHUNK = _B * _CH // _RPC
_NBUF = 10


def _f32_to_f16_bits_hi(y):
    u = jax.lax.bitcast_convert_type(y, jnp.int32)
    mag = u & jnp.int32(0x7FFFFFFF)
    rne = mag + jnp.int32(0x1000)
    t16 = (rne >> 13) - jnp.int32(0x1C000)
    sgn = (u >> 16) & jnp.int32(0x8000)
    h = jnp.where(mag >= jnp.int32(0x38800000), t16, jnp.int32(0)) | sgn
    return jax.lax.bitcast_convert_type(h << 16, jnp.float32)


def _scalar_col(tab_ref, ts_ref, base):
    l = jax.lax.broadcasted_iota(jnp.int32, (_CS, 1, 1), 0)
    col = jnp.full((_CS, 1, 1), tab_ref[ts_ref[base]], dtype=jnp.float32)
    for i in range(1, _CS):
        col = jnp.where(l == i, tab_ref[ts_ref[base + i]], col)
    return col


def _body(ts_ref, acp_ref, omacp_ref, lat_hbm, noi_hbm, out_hbm,
          lat_buf, noi_buf, out_buf, in_sems, out_sems):
    def load(k, slot):
        pltpu.make_async_copy(lat_hbm.at[pl.ds(k * _RPC, _RPC)],
                              lat_buf.at[slot], in_sems.at[slot, 0]).start()
        pltpu.make_async_copy(noi_hbm.at[pl.ds(k * _RPC, _RPC)],
                              noi_buf.at[slot], in_sems.at[slot, 1]).start()

    def wait_load(k, slot):
        pltpu.make_async_copy(lat_hbm.at[pl.ds(k * _RPC, _RPC)],
                              lat_buf.at[slot], in_sems.at[slot, 0]).wait()
        pltpu.make_async_copy(noi_hbm.at[pl.ds(k * _RPC, _RPC)],
                              noi_buf.at[slot], in_sems.at[slot, 1]).wait()

    def store(k, slot):
        pltpu.make_async_copy(out_buf.at[slot],
                              out_hbm.at[pl.ds(k * _RPC, _RPC)],
                              out_sems.at[slot]).start()

    def wait_store(k, slot):
        pltpu.make_async_copy(out_buf.at[slot],
                              out_hbm.at[pl.ds(k * _RPC, _RPC)],
                              out_sems.at[slot]).wait()

    for k in range(_NBUF):
        load(k, k)

    def step(k, carry):
        slot = jax.lax.rem(k, _NBUF)
        wait_load(k, slot)

        @pl.when(k >= _NBUF)
        def _():
            wait_store(k - _NBUF, slot)

        smp = k // (_CH // _RPC)
        s1 = acp_ref[ts_ref[smp]]
        s2 = omacp_ref[ts_ref[smp]]
        y = lat_buf[slot] * s1 + noi_buf[slot] * s2
        zf = _f32_to_f16_bits_hi(y)
        out_buf.bitcast(jnp.bfloat16).at[slot][...] = zf.astype(jnp.bfloat16)
        store(k, slot)

        @pl.when(k + _NBUF < _NCHUNK)
        def _():
            load(k + _NBUF, slot)

        return carry

    jax.lax.fori_loop(0, _NCHUNK, step, 0)
    for k in range(_NCHUNK - _NBUF, _NCHUNK):
        wait_store(k, k % _NBUF)


def kernel(latent, noise, timestep, sqrt_alphas_cum_prod, sqrt_one_minus_alphas_cum_prod):
    ts = timestep.astype(jnp.int32)
    acp = sqrt_alphas_cum_prod.astype(jnp.float16).astype(jnp.float32)
    omacp = sqrt_one_minus_alphas_cum_prod.astype(jnp.float16).astype(jnp.float32)
    lat3 = latent.reshape(_B * _CH, _W)
    noi3 = noise.reshape(_B * _CH, _W)

    grid_spec = pltpu.PrefetchScalarGridSpec(
        num_scalar_prefetch=3,
        grid=(1,),
        in_specs=[
            pl.BlockSpec(memory_space=pl.ANY),
            pl.BlockSpec(memory_space=pl.ANY),
        ],
        out_specs=pl.BlockSpec(memory_space=pl.ANY),
        scratch_shapes=[
            pltpu.VMEM((_NBUF, _RPC, _W), jnp.float32),
            pltpu.VMEM((_NBUF, _RPC, _W), jnp.float32),
            pltpu.VMEM((_NBUF, _RPC, _W), jnp.float16),
            pltpu.SemaphoreType.DMA((_NBUF, 2)),
            pltpu.SemaphoreType.DMA((_NBUF,)),
        ],
    )
    out = pl.pallas_call(
        _body,
        grid_spec=grid_spec,
        out_shape=jax.ShapeDtypeStruct((_B * _CH, _W), jnp.float16),
        compiler_params=pltpu.CompilerParams(
            vmem_limit_bytes=100 * 1024 * 1024,
        ),
    )(ts, acp, omacp, lat3, noi3)
    return out.reshape(_B, _C, _H, _W)


# speedup vs baseline: 1.0208x; 1.0208x over previous
import jax
import jax.numpy as jnp
from jax.experimental import pallas as pl
from jax.experimental.pallas import tpu as pltpu

_B, _C, _H, _W = 64, 3, 256, 256
_CH = _C * _H
_RPC = 384
_NCHUNK = _B * _CH // _RPC
_NBUF = 6


def _f32_to_f16_bits_hi(y):
    u = jax.lax.bitcast_convert_type(y, jnp.int32)
    mag = u & jnp.int32(0x7FFFFFFF)
    rne = mag + jnp.int32(0x1000)
    t16 = (rne >> 13) - jnp.int32(0x1C000)
    sgn = (u >> 16) & jnp.int32(0x8000)
    h = jnp.where(mag >= jnp.int32(0x38800000), t16, jnp.int32(0)) | sgn
    return jax.lax.bitcast_convert_type(h << 16, jnp.float32)


def _scalar_col(tab_ref, ts_ref, base):
    l = jax.lax.broadcasted_iota(jnp.int32, (_CS, 1, 1), 0)
    col = jnp.full((_CS, 1, 1), tab_ref[ts_ref[base]], dtype=jnp.float32)
    for i in range(1, _CS):
        col = jnp.where(l == i, tab_ref[ts_ref[base + i]], col)
    return col


def _body(ts_ref, acp_ref, omacp_ref, lat_hbm, noi_hbm, out_hbm,
          lat_buf, noi_buf, out_buf, in_sems, out_sems):
    def load(k, slot):
        pltpu.make_async_copy(lat_hbm.at[pl.ds(k * _RPC, _RPC)],
                              lat_buf.at[slot], in_sems.at[slot, 0]).start()
        pltpu.make_async_copy(noi_hbm.at[pl.ds(k * _RPC, _RPC)],
                              noi_buf.at[slot], in_sems.at[slot, 1]).start()

    def wait_load(k, slot):
        pltpu.make_async_copy(lat_hbm.at[pl.ds(k * _RPC, _RPC)],
                              lat_buf.at[slot], in_sems.at[slot, 0]).wait()
        pltpu.make_async_copy(noi_hbm.at[pl.ds(k * _RPC, _RPC)],
                              noi_buf.at[slot], in_sems.at[slot, 1]).wait()

    def store(k, slot):
        pltpu.make_async_copy(out_buf.at[slot],
                              out_hbm.at[pl.ds(k * _RPC, _RPC)],
                              out_sems.at[slot]).start()

    def wait_store(k, slot):
        pltpu.make_async_copy(out_buf.at[slot],
                              out_hbm.at[pl.ds(k * _RPC, _RPC)],
                              out_sems.at[slot]).wait()

    for k in range(_NBUF):
        load(k, k)

    def step(k, carry):
        slot = jax.lax.rem(k, _NBUF)
        wait_load(k, slot)

        @pl.when(k >= _NBUF)
        def _():
            wait_store(k - _NBUF, slot)

        smp = k // (_CH // _RPC)
        s1 = acp_ref[ts_ref[smp]]
        s2 = omacp_ref[ts_ref[smp]]
        y = lat_buf[slot] * s1 + noi_buf[slot] * s2
        zf = _f32_to_f16_bits_hi(y)
        out_buf.bitcast(jnp.bfloat16).at[slot][...] = zf.astype(jnp.bfloat16)
        store(k, slot)

        @pl.when(k + _NBUF < _NCHUNK)
        def _():
            load(k + _NBUF, slot)

        return carry

    jax.lax.fori_loop(0, _NCHUNK, step, 0)
    for k in range(_NCHUNK - _NBUF, _NCHUNK):
        wait_store(k, k % _NBUF)


def kernel(latent, noise, timestep, sqrt_alphas_cum_prod, sqrt_one_minus_alphas_cum_prod):
    ts = timestep.astype(jnp.int32)
    acp = sqrt_alphas_cum_prod.astype(jnp.float16).astype(jnp.float32)
    omacp = sqrt_one_minus_alphas_cum_prod.astype(jnp.float16).astype(jnp.float32)
    lat3 = latent.reshape(_B * _CH, _W)
    noi3 = noise.reshape(_B * _CH, _W)

    grid_spec = pltpu.PrefetchScalarGridSpec(
        num_scalar_prefetch=3,
        grid=(1,),
        in_specs=[
            pl.BlockSpec(memory_space=pl.ANY),
            pl.BlockSpec(memory_space=pl.ANY),
        ],
        out_specs=pl.BlockSpec(memory_space=pl.ANY),
        scratch_shapes=[
            pltpu.VMEM((_NBUF, _RPC, _W), jnp.float32),
            pltpu.VMEM((_NBUF, _RPC, _W), jnp.float32),
            pltpu.VMEM((_NBUF, _RPC, _W), jnp.float16),
            pltpu.SemaphoreType.DMA((_NBUF, 2)),
            pltpu.SemaphoreType.DMA((_NBUF,)),
        ],
    )
    out = pl.pallas_call(
        _body,
        grid_spec=grid_spec,
        out_shape=jax.ShapeDtypeStruct((_B * _CH, _W), jnp.float16),
        compiler_params=pltpu.CompilerParams(
            vmem_limit_bytes=100 * 1024 * 1024,
        ),
    )(ts, acp, omacp, lat3, noi3)
    return out.reshape(_B, _C, _H, _W)


# speedup vs baseline: 1.0525x; 1.0311x over previous
import jax
import jax.numpy as jnp
from jax.experimental import pallas as pl
from jax.experimental.pallas import tpu as pltpu

_B, _C, _H, _W = 64, 3, 256, 256
_CH = _C * _H
_RPC = 384
_NCHUNK = _B * _CH // _RPC
_NBUF = 10


def _f32_to_f16_bits_hi(y):
    u = jax.lax.bitcast_convert_type(y, jnp.int32)
    mag = u & jnp.int32(0x7FFFFFFF)
    rne = mag + jnp.int32(0x1000)
    t16 = (rne >> 13) - jnp.int32(0x1C000)
    sgn = (u >> 16) & jnp.int32(0x8000)
    h = jnp.where(mag >= jnp.int32(0x38800000), t16, jnp.int32(0)) | sgn
    return jax.lax.bitcast_convert_type(h << 16, jnp.float32)


def _scalar_col(tab_ref, ts_ref, base):
    l = jax.lax.broadcasted_iota(jnp.int32, (_CS, 1, 1), 0)
    col = jnp.full((_CS, 1, 1), tab_ref[ts_ref[base]], dtype=jnp.float32)
    for i in range(1, _CS):
        col = jnp.where(l == i, tab_ref[ts_ref[base + i]], col)
    return col


def _body(ts_ref, acp_ref, omacp_ref, lat_hbm, noi_hbm, out_hbm,
          lat_buf, noi_buf, out_buf, in_sems, out_sems):
    def load(k, slot):
        pltpu.make_async_copy(lat_hbm.at[pl.ds(k * _RPC, _RPC)],
                              lat_buf.at[slot], in_sems.at[slot, 0]).start()
        pltpu.make_async_copy(noi_hbm.at[pl.ds(k * _RPC, _RPC)],
                              noi_buf.at[slot], in_sems.at[slot, 1]).start()

    def wait_load(k, slot):
        pltpu.make_async_copy(lat_hbm.at[pl.ds(k * _RPC, _RPC)],
                              lat_buf.at[slot], in_sems.at[slot, 0]).wait()
        pltpu.make_async_copy(noi_hbm.at[pl.ds(k * _RPC, _RPC)],
                              noi_buf.at[slot], in_sems.at[slot, 1]).wait()

    def store(k, slot):
        pltpu.make_async_copy(out_buf.at[slot],
                              out_hbm.at[pl.ds(k * _RPC, _RPC)],
                              out_sems.at[slot]).start()

    def wait_store(k, slot):
        pltpu.make_async_copy(out_buf.at[slot],
                              out_hbm.at[pl.ds(k * _RPC, _RPC)],
                              out_sems.at[slot]).wait()

    for k in range(_NBUF):
        load(k, k)

    def step(k, carry):
        slot = jax.lax.rem(k, _NBUF)
        wait_load(k, slot)

        @pl.when(k >= _NBUF)
        def _():
            wait_store(k - _NBUF, slot)

        smp = k // (_CH // _RPC)
        s1 = acp_ref[ts_ref[smp]]
        s2 = omacp_ref[ts_ref[smp]]
        y = lat_buf[slot] * s1 + noi_buf[slot] * s2
        zf = _f32_to_f16_bits_hi(y)
        out_buf.bitcast(jnp.bfloat16).at[slot][...] = zf.astype(jnp.bfloat16)
        store(k, slot)

        @pl.when(k + _NBUF < _NCHUNK)
        def _():
            load(k + _NBUF, slot)

        return carry

    jax.lax.fori_loop(0, _NCHUNK, step, 0)
    for k in range(_NCHUNK - _NBUF, _NCHUNK):
        wait_store(k, k % _NBUF)


def kernel(latent, noise, timestep, sqrt_alphas_cum_prod, sqrt_one_minus_alphas_cum_prod):
    ts = timestep.astype(jnp.int32)
    acp = sqrt_alphas_cum_prod.astype(jnp.float16).astype(jnp.float32)
    omacp = sqrt_one_minus_alphas_cum_prod.astype(jnp.float16).astype(jnp.float32)
    lat3 = latent.reshape(_B * _CH, _W)
    noi3 = noise.reshape(_B * _CH, _W)

    grid_spec = pltpu.PrefetchScalarGridSpec(
        num_scalar_prefetch=3,
        grid=(1,),
        in_specs=[
            pl.BlockSpec(memory_space=pl.ANY),
            pl.BlockSpec(memory_space=pl.ANY),
        ],
        out_specs=pl.BlockSpec(memory_space=pl.ANY),
        scratch_shapes=[
            pltpu.VMEM((_NBUF, _RPC, _W), jnp.float32),
            pltpu.VMEM((_NBUF, _RPC, _W), jnp.float32),
            pltpu.VMEM((_NBUF, _RPC, _W), jnp.float16),
            pltpu.SemaphoreType.DMA((_NBUF, 2)),
            pltpu.SemaphoreType.DMA((_NBUF,)),
        ],
    )
    out = pl.pallas_call(
        _body,
        grid_spec=grid_spec,
        out_shape=jax.ShapeDtypeStruct((_B * _CH, _W), jnp.float16),
        compiler_params=pltpu.CompilerParams(
            vmem_limit_bytes=100 * 1024 * 1024,
        ),
    )(ts, acp, omacp, lat3, noi3)
    return out.reshape(_B, _C, _H, _W)


# speedup vs baseline: 1.0567x; 1.0040x over previous
import jax
import jax.numpy as jnp
from jax.experimental import pallas as pl
from jax.experimental.pallas import tpu as pltpu

_B, _C, _H, _W = 64, 3, 256, 256
_CH = _C * _H
_RPC = 384
_NCHUNK = _B * _CH // _RPC
_NBUF = 8


def _f32_to_f16_bits_hi(y):
    u = jax.lax.bitcast_convert_type(y, jnp.int32)
    mag = u & jnp.int32(0x7FFFFFFF)
    t16 = (mag + jnp.int32(0x1000)) >> 13
    t16 = jnp.where(mag >= jnp.int32(0x00800000), t16, jnp.int32(0))
    h = (t16 << 16) | (u & jnp.int32(-0x80000000))
    return jax.lax.bitcast_convert_type(h, jnp.float32)


def _scalar_col(tab_ref, ts_ref, base):
    l = jax.lax.broadcasted_iota(jnp.int32, (_CS, 1, 1), 0)
    col = jnp.full((_CS, 1, 1), tab_ref[ts_ref[base]], dtype=jnp.float32)
    for i in range(1, _CS):
        col = jnp.where(l == i, tab_ref[ts_ref[base + i]], col)
    return col


def _body(ts_ref, acp_ref, omacp_ref, lat_hbm, noi_hbm, out_hbm,
          lat_buf, noi_buf, out_buf, in_sems, out_sems):
    def load(k, slot):
        pltpu.make_async_copy(lat_hbm.at[pl.ds(k * _RPC, _RPC)],
                              lat_buf.at[slot], in_sems.at[slot, 0]).start()
        pltpu.make_async_copy(noi_hbm.at[pl.ds(k * _RPC, _RPC)],
                              noi_buf.at[slot], in_sems.at[slot, 1]).start()

    def wait_load(k, slot):
        pltpu.make_async_copy(lat_hbm.at[pl.ds(k * _RPC, _RPC)],
                              lat_buf.at[slot], in_sems.at[slot, 0]).wait()
        pltpu.make_async_copy(noi_hbm.at[pl.ds(k * _RPC, _RPC)],
                              noi_buf.at[slot], in_sems.at[slot, 1]).wait()

    def store(k, slot):
        pltpu.make_async_copy(out_buf.at[slot],
                              out_hbm.at[pl.ds(k * _RPC, _RPC)],
                              out_sems.at[slot]).start()

    def wait_store(k, slot):
        pltpu.make_async_copy(out_buf.at[slot],
                              out_hbm.at[pl.ds(k * _RPC, _RPC)],
                              out_sems.at[slot]).wait()

    for k in range(_NBUF):
        load(k, k)

    def step(k, carry):
        slot = jax.lax.rem(k, _NBUF)
        wait_load(k, slot)

        @pl.when(k >= _NBUF)
        def _():
            wait_store(k - _NBUF, slot)

        smp = k // (_CH // _RPC)
        s1 = acp_ref[ts_ref[smp]]
        s2 = omacp_ref[ts_ref[smp]]
        y = lat_buf[slot] * s1 + noi_buf[slot] * s2
        zf = _f32_to_f16_bits_hi(y)
        out_buf.bitcast(jnp.bfloat16).at[slot][...] = zf.astype(jnp.bfloat16)
        store(k, slot)

        @pl.when(k + _NBUF < _NCHUNK)
        def _():
            load(k + _NBUF, slot)

        return carry

    jax.lax.fori_loop(0, _NCHUNK, step, 0)
    for k in range(_NCHUNK - _NBUF, _NCHUNK):
        wait_store(k, k % _NBUF)


def kernel(latent, noise, timestep, sqrt_alphas_cum_prod, sqrt_one_minus_alphas_cum_prod):
    ts = timestep.astype(jnp.int32)
    acp = sqrt_alphas_cum_prod.astype(jnp.float16).astype(jnp.float32) * jnp.float32(2.0 ** -112)
    omacp = sqrt_one_minus_alphas_cum_prod.astype(jnp.float16).astype(jnp.float32) * jnp.float32(2.0 ** -112)
    lat3 = latent.reshape(_B * _CH, _W)
    noi3 = noise.reshape(_B * _CH, _W)

    grid_spec = pltpu.PrefetchScalarGridSpec(
        num_scalar_prefetch=3,
        grid=(1,),
        in_specs=[
            pl.BlockSpec(memory_space=pl.ANY),
            pl.BlockSpec(memory_space=pl.ANY),
        ],
        out_specs=pl.BlockSpec(memory_space=pl.ANY),
        scratch_shapes=[
            pltpu.VMEM((_NBUF, _RPC, _W), jnp.float32),
            pltpu.VMEM((_NBUF, _RPC, _W), jnp.float32),
            pltpu.VMEM((_NBUF, _RPC, _W), jnp.float16),
            pltpu.SemaphoreType.DMA((_NBUF, 2)),
            pltpu.SemaphoreType.DMA((_NBUF,)),
        ],
    )
    out = pl.pallas_call(
        _body,
        grid_spec=grid_spec,
        out_shape=jax.ShapeDtypeStruct((_B * _CH, _W), jnp.float16),
        compiler_params=pltpu.CompilerParams(
            vmem_limit_bytes=100 * 1024 * 1024,
        ),
    )(ts, acp, omacp, lat3, noi3)
    return out.reshape(_B, _C, _H, _W)
